# baseline (speedup 1.0000x reference)
import functools

import jax
import jax.numpy as jnp
from jax import lax
from jax.experimental import pallas as pl
from jax.experimental.pallas import tpu as pltpu
from jax.experimental.pallas import tpu_sc as plsc

VOCAB = 1000000
DIM = 16
BATCH = 16384
LANE = 128

NUM_CORES = 2
NUM_SUBCORES = 16
NW = NUM_CORES * NUM_SUBCORES
BPW = BATCH // NW
G = 8
NG = BPW // G
NGT = 2 * NG
NSLOT = 3
L = 16


def _fire_group(wt_hbm, idxv, slabs_slot, sem, gg, half):
    kv = idxv[pl.ds((gg - half) * G, 2 * G)]
    for b in range(G):
        k = kv[b + half * G]
        off = pl.multiple_of(lax.shift_right_logical(k, 7) * LANE, LANE)
        pltpu.async_copy(wt_hbm.at[pl.ds(0, 8), pl.ds(off, LANE)],
                         slabs_slot.at[b, pl.ds(0, 8)], sem)
        pltpu.async_copy(wt_hbm.at[pl.ds(8, 8), pl.ds(off, LANE)],
                         slabs_slot.at[b, pl.ds(8, 8)], sem)


def _drain_extract_fire(wt_hbm, idxv, slabs, blk, sem, gg, slot, half):
    for b in range(G):
        pltpu.make_async_copy(wt_hbm.at[pl.ds(0, 8), pl.ds(0, LANE)],
                              slabs.at[slot, b, pl.ds(0, 8)], sem).wait()
        pltpu.make_async_copy(wt_hbm.at[pl.ds(8, 8), pl.ds(0, LANE)],
                              slabs.at[slot, b, pl.ds(8, 8)], sem).wait()
    kv = idxv[pl.ds((gg - half) * G, 2 * G)]
    cv = lax.bitwise_and(kv, LANE - 1)
    r_vec = lax.iota(jnp.int32, L)
    slot_vec = jnp.full((L,), slot, jnp.int32)
    for b in range(G):
        val = plsc.load_gather(
            slabs, [slot_vec, jnp.full((L,), b, jnp.int32), r_vec,
                    jnp.full((L,), cv[b + half * G], jnp.int32)])
        plsc.store_scatter(blk, [r_vec, jnp.full((L,), gg * G + b, jnp.int32)],
                           val)

    @pl.when(gg + NSLOT < NG)
    def _():
        _fire_group(wt_hbm, idxv, slabs.at[slot], sem, gg + NSLOT,
                    (half + NSLOT) % 2)


@functools.partial(
    pl.kernel,
    mesh=plsc.VectorSubcoreMesh(core_axis_name="c", subcore_axis_name="s"),
    out_type=[
        jax.ShapeDtypeStruct((DIM, BATCH), jnp.float32),
        jax.ShapeDtypeStruct((DIM, BATCH), jnp.float32),
    ],
    scratch_types=[
        pltpu.VMEM((BPW,), jnp.int32),
        pltpu.VMEM((BPW,), jnp.int32),
        pltpu.VMEM((NSLOT, G, DIM, LANE), jnp.float32),
        pltpu.VMEM((NSLOT, G, DIM, LANE), jnp.float32),
        pltpu.VMEM((DIM, BPW), jnp.float32),
        pltpu.VMEM((DIM, BPW), jnp.float32),
        pltpu.SemaphoreType.DMA,
        pltpu.SemaphoreType.DMA,
        pltpu.SemaphoreType.DMA,
        pltpu.SemaphoreType.DMA,
        pltpu.SemaphoreType.DMA,
        pltpu.SemaphoreType.DMA,
    ],
    compiler_params=pltpu.CompilerParams(needs_layout_passes=False),
)
def _emb_lookup(user_hbm, item_hbm, wtu_hbm, wti_hbm, otu_hbm, oti_hbm,
                idxv_u, idxv_i, slabs_u, slabs_i, blk_u, blk_i,
                sem_u0, sem_u1, sem_u2, sem_i0, sem_i1, sem_i2):
    wid = lax.axis_index("s") * NUM_CORES + lax.axis_index("c")
    base = wid * BPW

    pltpu.sync_copy(user_hbm.at[wid], idxv_u)
    pltpu.sync_copy(item_hbm.at[wid], idxv_i)

    _fire_group(wtu_hbm, idxv_u, slabs_u.at[0], sem_u0, 0, 0)
    _fire_group(wti_hbm, idxv_i, slabs_i.at[0], sem_i0, 0, 0)
    _fire_group(wtu_hbm, idxv_u, slabs_u.at[1], sem_u1, 1, 1)
    _fire_group(wti_hbm, idxv_i, slabs_i.at[1], sem_i1, 1, 1)
    _fire_group(wtu_hbm, idxv_u, slabs_u.at[2], sem_u2, 2, 0)
    _fire_group(wti_hbm, idxv_i, slabs_i.at[2], sem_i2, 2, 0)

    def body(j, _):
        gg = lax.div(j, 2)
        m6 = lax.rem(gg, 6)
        is_user = lax.rem(j, 2) == 0
        cases = [(0, 0, 0), (1, 1, 1), (2, 2, 0), (3, 0, 1), (4, 1, 0),
                 (5, 2, 1)]
        sems_u = (sem_u0, sem_u1, sem_u2)
        sems_i = (sem_i0, sem_i1, sem_i2)
        for m, slot, half in cases:
            @pl.when(jnp.logical_and(is_user, m6 == m))
            def _(slot=slot, half=half):
                _drain_extract_fire(wtu_hbm, idxv_u, slabs_u, blk_u,
                                    sems_u[slot], gg, slot, half)

            @pl.when(jnp.logical_and(jnp.logical_not(is_user), m6 == m))
            def _(slot=slot, half=half):
                _drain_extract_fire(wti_hbm, idxv_i, slabs_i, blk_i,
                                    sems_i[slot], gg, slot, half)

        return _

    lax.fori_loop(0, NGT, body, None)

    pltpu.sync_copy(blk_u, otu_hbm.at[:, pl.ds(base, BPW)])
    pltpu.sync_copy(blk_i, oti_hbm.at[:, pl.ds(base, BPW)])


def kernel(user, item, W_user, W_item):
    u = user.astype(jnp.int32).reshape(NW, BPW)
    it = item.astype(jnp.int32).reshape(NW, BPW)
    out_u_t, out_i_t = _emb_lookup(u, it, W_user.T, W_item.T)
    return out_u_t.T, out_i_t.T


# speedup vs baseline: 3.0126x; 3.0126x over previous
import functools

import jax
import jax.numpy as jnp
from jax import lax
from jax.experimental import pallas as pl
from jax.experimental.pallas import tpu as pltpu
from jax.experimental.pallas import tpu_sc as plsc

VOCAB = 1000000
DIM = 16
BATCH = 16384
LANE = 128

NUM_CORES = 2
NUM_SUBCORES = 16
NW = NUM_CORES * NUM_SUBCORES
BPW = BATCH // NW
G = 8
NG = BPW // G
NGT = 2 * NG
NSLOT = 3
L = 16


def _fire_group(wt_hbm, idxv, slabs_slot, sem, gg, half):
    kv = idxv[pl.ds((gg - half) * G, 2 * G)]
    for b in range(G):
        k = kv[b + half * G]
        off = pl.multiple_of(lax.shift_right_logical(k, 7) * LANE, LANE)
        pltpu.async_copy(wt_hbm.at[:, pl.ds(off, LANE)],
                         slabs_slot.at[b], sem)


def _drain_extract_fire(wt_hbm, idxv, slabs, blk, sem, gg, slot, half):
    for b in range(G):
        pltpu.make_async_copy(wt_hbm.at[:, pl.ds(0, LANE)],
                              slabs.at[slot, b], sem).wait()
    kv = idxv[pl.ds((gg - half) * G, 2 * G)]
    cv = lax.bitwise_and(kv, LANE - 1)
    r_vec = lax.iota(jnp.int32, L)
    slot_vec = jnp.full((L,), slot, jnp.int32)
    for b in range(G):
        val = plsc.load_gather(
            slabs, [slot_vec, jnp.full((L,), b, jnp.int32), r_vec,
                    jnp.full((L,), cv[b + half * G], jnp.int32)])
        plsc.store_scatter(blk, [r_vec, jnp.full((L,), gg * G + b, jnp.int32)],
                           val)

    @pl.when(gg + NSLOT < NG)
    def _():
        _fire_group(wt_hbm, idxv, slabs.at[slot], sem, gg + NSLOT,
                    (half + NSLOT) % 2)


@functools.partial(
    pl.kernel,
    mesh=plsc.VectorSubcoreMesh(core_axis_name="c", subcore_axis_name="s"),
    out_type=[
        jax.ShapeDtypeStruct((DIM, BATCH), jnp.float32),
        jax.ShapeDtypeStruct((DIM, BATCH), jnp.float32),
    ],
    scratch_types=[
        pltpu.VMEM((BPW,), jnp.int32),
        pltpu.VMEM((BPW,), jnp.int32),
        pltpu.VMEM((NSLOT, G, DIM, LANE), jnp.float32),
        pltpu.VMEM((NSLOT, G, DIM, LANE), jnp.float32),
        pltpu.VMEM((DIM, BPW), jnp.float32),
        pltpu.VMEM((DIM, BPW), jnp.float32),
        pltpu.SemaphoreType.DMA,
        pltpu.SemaphoreType.DMA,
        pltpu.SemaphoreType.DMA,
        pltpu.SemaphoreType.DMA,
        pltpu.SemaphoreType.DMA,
        pltpu.SemaphoreType.DMA,
    ],
    compiler_params=pltpu.CompilerParams(needs_layout_passes=False),
)
def _emb_lookup(user_hbm, item_hbm, wtu_hbm, wti_hbm, otu_hbm, oti_hbm,
                idxv_u, idxv_i, slabs_u, slabs_i, blk_u, blk_i,
                sem_u0, sem_u1, sem_u2, sem_i0, sem_i1, sem_i2):
    wid = lax.axis_index("s") * NUM_CORES + lax.axis_index("c")
    base = wid * BPW

    pltpu.sync_copy(user_hbm.at[wid], idxv_u)
    pltpu.sync_copy(item_hbm.at[wid], idxv_i)

    _fire_group(wtu_hbm, idxv_u, slabs_u.at[0], sem_u0, 0, 0)
    _fire_group(wti_hbm, idxv_i, slabs_i.at[0], sem_i0, 0, 0)
    _fire_group(wtu_hbm, idxv_u, slabs_u.at[1], sem_u1, 1, 1)
    _fire_group(wti_hbm, idxv_i, slabs_i.at[1], sem_i1, 1, 1)
    _fire_group(wtu_hbm, idxv_u, slabs_u.at[2], sem_u2, 2, 0)
    _fire_group(wti_hbm, idxv_i, slabs_i.at[2], sem_i2, 2, 0)

    def body(j, _):
        gg = lax.div(j, 2)
        m6 = lax.rem(gg, 6)
        is_user = lax.rem(j, 2) == 0
        cases = [(0, 0, 0), (1, 1, 1), (2, 2, 0), (3, 0, 1), (4, 1, 0),
                 (5, 2, 1)]
        sems_u = (sem_u0, sem_u1, sem_u2)
        sems_i = (sem_i0, sem_i1, sem_i2)
        for m, slot, half in cases:
            @pl.when(jnp.logical_and(is_user, m6 == m))
            def _(slot=slot, half=half):
                _drain_extract_fire(wtu_hbm, idxv_u, slabs_u, blk_u,
                                    sems_u[slot], gg, slot, half)

            @pl.when(jnp.logical_and(jnp.logical_not(is_user), m6 == m))
            def _(slot=slot, half=half):
                _drain_extract_fire(wti_hbm, idxv_i, slabs_i, blk_i,
                                    sems_i[slot], gg, slot, half)

        return _

    lax.fori_loop(0, NGT, body, None)

    pltpu.sync_copy(blk_u, otu_hbm.at[:, pl.ds(base, BPW)])
    pltpu.sync_copy(blk_i, oti_hbm.at[:, pl.ds(base, BPW)])


def kernel(user, item, W_user, W_item):
    u = user.astype(jnp.int32).reshape(NW, BPW)
    it = item.astype(jnp.int32).reshape(NW, BPW)
    out_u_t, out_i_t = _emb_lookup(u, it, W_user.T, W_item.T)
    return out_u_t.T, out_i_t.T
